# Initial kernel scaffold; baseline (speedup 1.0000x reference)
import jax
import jax.numpy as jnp
from jax.experimental import pallas as pl


def kernel(q, k, v, gating_input, gate_W, gate_b):
    raise NotImplementedError("write your pallas kernel here")


# speedup vs baseline: 2148.4978x; 2148.4978x over previous
import functools

import jax
import jax.numpy as jnp
from jax import lax
from jax.experimental import pallas as pl
from jax.experimental.pallas import tpu as pltpu

_TOPK = 8


def _gating_kernel(gin_ref, w_ref, b_ref, idx_ref, loss_ref):
    g = gin_ref[...]
    B = g.shape[0]
    H = w_ref.shape[0]
    gmean = jnp.mean(g, axis=1)
    logits = lax.dot_general(
        gmean, w_ref[...], (((1,), (1,)), ((), ())),
        preferred_element_type=jnp.float32) + b_ref[...]
    m = jnp.max(logits, axis=1, keepdims=True)
    e = jnp.exp(logits - m)
    sm = e / jnp.sum(e, axis=1, keepdims=True)

    iota = lax.broadcasted_iota(jnp.int32, (B, H), 1)
    vals = sm
    mask_acc = jnp.zeros((B, H), jnp.float32)
    cols = []
    for _ in range(_TOPK):
        mx = jnp.max(vals, axis=1, keepdims=True)
        is_mx = vals == mx
        idx = jnp.min(jnp.where(is_mx, iota, H), axis=1, keepdims=True)
        one_hot = (iota == idx).astype(jnp.float32)
        mask_acc = mask_acc + one_hot
        vals = jnp.where(one_hot > 0.0, -jnp.inf, vals)
        cols.append(idx)
    top_idx = jnp.concatenate(cols, axis=1)
    idx_ref[...] = top_idx.astype(jnp.int32)

    tokens_per_expert = jnp.sum(mask_acc, axis=0, keepdims=True)
    router_prob = jnp.mean(sm, axis=0, keepdims=True)
    density = tokens_per_expert / (jnp.sum(tokens_per_expert) + 1e-6)
    loss_ref[...] = H * jnp.sum(router_prob * density, axis=1, keepdims=True)


def _attn_kernel(idx_ref, q_ref, k_ref, v_ref, o_ref, *, scale):
    q = q_ref[0, 0] * scale
    k = k_ref[0, 0]
    v = v_ref[0, 0]
    s = lax.dot_general(q, k, (((1,), (1,)), ((), ())),
                        preferred_element_type=jnp.float32)
    m = jnp.max(s, axis=1, keepdims=True)
    p = jnp.exp(s - m)
    p = p / jnp.sum(p, axis=1, keepdims=True)
    o_ref[0] = jnp.dot(p, v, preferred_element_type=jnp.float32)


def kernel(q, k, v, gating_input, gate_W, gate_b):
    B, H, N, D = q.shape
    K = _TOPK
    BQ = 512

    top_idx, lb_loss = pl.pallas_call(
        _gating_kernel,
        out_shape=(
            jax.ShapeDtypeStruct((B, K), jnp.int32),
            jax.ShapeDtypeStruct((1, 1), jnp.float32),
        ),
    )(gating_input, gate_W, gate_b.reshape(1, H))

    grid = (B, K, N // BQ)
    attn = pl.pallas_call(
        functools.partial(_attn_kernel, scale=float(D) ** -0.5),
        grid_spec=pltpu.PrefetchScalarGridSpec(
            num_scalar_prefetch=1,
            grid=grid,
            in_specs=[
                pl.BlockSpec((1, 1, BQ, D), lambda b, kk, qi, idx: (b, idx[b, kk], qi, 0)),
                pl.BlockSpec((1, 1, N, D), lambda b, kk, qi, idx: (b, idx[b, kk], 0, 0)),
                pl.BlockSpec((1, 1, N, D), lambda b, kk, qi, idx: (b, idx[b, kk], 0, 0)),
            ],
            out_specs=pl.BlockSpec((1, BQ, D), lambda b, kk, qi, idx: (b, qi, kk)),
        ),
        out_shape=jax.ShapeDtypeStruct((B, N, K * D), jnp.float32),
    )(top_idx, q, k, v)

    return attn, lb_loss.reshape(())


# speedup vs baseline: 2240.8389x; 1.0430x over previous
import functools

import jax
import jax.numpy as jnp
from jax import lax
from jax.experimental import pallas as pl
from jax.experimental.pallas import tpu as pltpu

_TOPK = 8


def _gating_kernel(gin_ref, w_ref, b_ref, idx_ref, loss_ref):
    g = gin_ref[...]
    B = g.shape[0]
    H = w_ref.shape[0]
    gmean = jnp.mean(g, axis=1)
    logits = lax.dot_general(
        gmean, w_ref[...], (((1,), (1,)), ((), ())),
        preferred_element_type=jnp.float32) + b_ref[...]
    m = jnp.max(logits, axis=1, keepdims=True)
    e = jnp.exp(logits - m)
    sm = e / jnp.sum(e, axis=1, keepdims=True)

    iota = lax.broadcasted_iota(jnp.int32, (B, H), 1)
    vals = sm
    mask_acc = jnp.zeros((B, H), jnp.float32)
    cols = []
    for _ in range(_TOPK):
        mx = jnp.max(vals, axis=1, keepdims=True)
        is_mx = vals == mx
        idx = jnp.min(jnp.where(is_mx, iota, H), axis=1, keepdims=True)
        one_hot = (iota == idx).astype(jnp.float32)
        mask_acc = mask_acc + one_hot
        vals = jnp.where(one_hot > 0.0, -jnp.inf, vals)
        cols.append(idx)
    top_idx = jnp.concatenate(cols, axis=1)
    idx_ref[...] = top_idx.astype(jnp.int32)

    tokens_per_expert = jnp.sum(mask_acc, axis=0, keepdims=True)
    router_prob = jnp.mean(sm, axis=0, keepdims=True)
    density = tokens_per_expert / (jnp.sum(tokens_per_expert) + 1e-6)
    loss_ref[...] = H * jnp.sum(router_prob * density, axis=1, keepdims=True)


def _attn_kernel(idx_ref, q_ref, k_ref, v_ref, o_ref, *, scale):
    q = q_ref[0, 0] * scale
    k = k_ref[0, 0]
    v = v_ref[0, 0]
    s = lax.dot_general(q, k, (((1,), (1,)), ((), ())),
                        preferred_element_type=jnp.float32)
    m = jnp.max(s, axis=1, keepdims=True)
    p = jnp.exp(s - m)
    r = jnp.sum(p, axis=1, keepdims=True)
    acc = jnp.dot(p.astype(jnp.bfloat16), v.astype(jnp.bfloat16),
                  preferred_element_type=jnp.float32)
    o_ref[0] = acc / r


def kernel(q, k, v, gating_input, gate_W, gate_b):
    B, H, N, D = q.shape
    K = _TOPK
    BQ = 512

    top_idx, lb_loss = pl.pallas_call(
        _gating_kernel,
        out_shape=(
            jax.ShapeDtypeStruct((B, K), jnp.int32),
            jax.ShapeDtypeStruct((1, 1), jnp.float32),
        ),
    )(gating_input, gate_W, gate_b.reshape(1, H))

    grid = (B, K, N // BQ)
    attn = pl.pallas_call(
        functools.partial(_attn_kernel, scale=float(D) ** -0.5),
        grid_spec=pltpu.PrefetchScalarGridSpec(
            num_scalar_prefetch=1,
            grid=grid,
            in_specs=[
                pl.BlockSpec((1, 1, BQ, D), lambda b, kk, qi, idx: (b, idx[b, kk], qi, 0)),
                pl.BlockSpec((1, 1, N, D), lambda b, kk, qi, idx: (b, idx[b, kk], 0, 0)),
                pl.BlockSpec((1, 1, N, D), lambda b, kk, qi, idx: (b, idx[b, kk], 0, 0)),
            ],
            out_specs=pl.BlockSpec((1, BQ, D), lambda b, kk, qi, idx: (b, qi, kk)),
        ),
        out_shape=jax.ShapeDtypeStruct((B, N, K * D), jnp.float32),
    )(top_idx, q, k, v)

    return attn, lb_loss.reshape(())
